# baseline (speedup 1.0000x reference)
import functools

import jax
import jax.numpy as jnp
from jax import lax
from jax.experimental import pallas as pl
from jax.experimental.pallas import tpu as pltpu, tpu_sc as plsc

_N, _C = 100000, 240
_NC, _NS = 2, 16
_NW = _NC * _NS
_RPW = _N // _NW
_RB = 125
_NB = _RPW // _RB

_mesh = plsc.VectorSubcoreMesh(core_axis_name="c", subcore_axis_name="s")


@functools.partial(
    pl.kernel,
    out_type=jax.ShapeDtypeStruct((_N, _C), jnp.float32),
    mesh=_mesh,
    scratch_types=[pltpu.VMEM((_RB, _C), jnp.float32)],
    compiler_params=pltpu.CompilerParams(use_tc_tiling_on_sc=False),
)
def _sc_permute(x_hbm, o_hbm, buf):
    wid = lax.axis_index("s") * _NC + lax.axis_index("c")
    base = wid * _RPW

    def step(g, carry):
        r0 = base + g * _RB
        rows = pl.ds(r0, _RB)
        pltpu.sync_copy(x_hbm.at[rows, pl.ds(96, 64)], buf.at[:, pl.ds(0, 64)])
        pltpu.sync_copy(x_hbm.at[rows, pl.ds(0, 96)], buf.at[:, pl.ds(64, 96)])
        pltpu.sync_copy(x_hbm.at[rows, pl.ds(160, 80)], buf.at[:, pl.ds(160, 80)])
        pltpu.sync_copy(buf, o_hbm.at[rows])
        return carry

    lax.fori_loop(0, _NB, step, 0)


def kernel(x):
    return _sc_permute(x)


# speedup vs baseline: 1.0463x; 1.0463x over previous
import functools

import jax
import jax.numpy as jnp
from jax import lax
from jax.experimental import pallas as pl
from jax.experimental.pallas import tpu as pltpu, tpu_sc as plsc

_N, _C = 100000, 240
_NC, _NS = 2, 16
_NW = _NC * _NS
_RPW = _N // _NW
_RB = 125
_NB = _RPW // _RB
_NBUF = 4

_mesh = plsc.VectorSubcoreMesh(core_axis_name="c", subcore_axis_name="s")


@functools.partial(
    pl.kernel,
    out_type=jax.ShapeDtypeStruct((_N, _C), jnp.float32),
    mesh=_mesh,
    scratch_types=(
        [pltpu.VMEM((_RB, _C), jnp.float32) for _ in range(_NBUF)]
        + [pltpu.SemaphoreType.DMA for _ in range(2 * _NBUF)]
    ),
    compiler_params=pltpu.CompilerParams(use_tc_tiling_on_sc=False),
)
def _sc_permute(x_hbm, o_hbm, *scratch):
    bufs = scratch[:_NBUF]
    isems = scratch[_NBUF:2 * _NBUF]
    osems = scratch[2 * _NBUF:]
    wid = lax.axis_index("s") * _NC + lax.axis_index("c")
    base = wid * _RPW

    in_descs = [None] * _NB
    out_descs = [None] * _NB

    def start_gather(g):
        b = g % _NBUF
        rows = pl.ds(base + g * _RB, _RB)
        in_descs[g] = [
            pltpu.async_copy(x_hbm.at[rows, pl.ds(96, 64)],
                             bufs[b].at[:, pl.ds(0, 64)], isems[b]),
            pltpu.async_copy(x_hbm.at[rows, pl.ds(0, 96)],
                             bufs[b].at[:, pl.ds(64, 96)], isems[b]),
            pltpu.async_copy(x_hbm.at[rows, pl.ds(160, 80)],
                             bufs[b].at[:, pl.ds(160, 80)], isems[b]),
        ]

    def start_write(g):
        b = g % _NBUF
        rows = pl.ds(base + g * _RB, _RB)
        out_descs[g] = pltpu.async_copy(bufs[b], o_hbm.at[rows], osems[b])

    start_gather(0)
    if _NB > 1:
        start_gather(1)
    for g in range(_NB):
        for d in in_descs[g]:
            d.wait()
        start_write(g)
        ng = g + 2
        if ng < _NB:
            if ng >= _NBUF:
                out_descs[ng - _NBUF].wait()
            start_gather(ng)
    for g in range(max(0, _NB - _NBUF), _NB):
        out_descs[g].wait()


def kernel(x):
    return _sc_permute(x)


# speedup vs baseline: 4.8036x; 4.5909x over previous
import jax
import jax.numpy as jnp
from jax.experimental import pallas as pl

_N, _C = 100000, 240
_RB = 4000


def _permute_body(x_ref, o_ref):
    x = x_ref[...]
    o_ref[:, 0:64] = x[:, 96:160]
    o_ref[:, 64:160] = x[:, 0:96]
    o_ref[:, 160:240] = x[:, 160:240]


def kernel(x):
    return pl.pallas_call(
        _permute_body,
        grid=(_N // _RB,),
        in_specs=[pl.BlockSpec((_RB, _C), lambda i: (i, 0))],
        out_specs=pl.BlockSpec((_RB, _C), lambda i: (i, 0)),
        out_shape=jax.ShapeDtypeStruct((_N, _C), x.dtype),
    )(x)
